# baseline (speedup 1.0000x reference)
import functools

import jax
import jax.numpy as jnp
from jax import lax
from jax.experimental import pallas as pl
from jax.experimental.pallas import tpu as pltpu
from jax.experimental.pallas import tpu_sc as plsc

SEQ = 200
BATCH = 1024
DIM = 64
N = SEQ * BATCH
NC = 2
NS = 16
NW = NC * NS
PER_W = N // NW
G = 128
NG = PER_W // G
NBUF = 10
NBLK = NG // NBUF


@functools.partial(
    pl.kernel,
    out_type=jax.ShapeDtypeStruct((N, DIM), jnp.float32),
    mesh=plsc.VectorSubcoreMesh(core_axis_name="c", subcore_axis_name="s"),
    compiler_params=pltpu.CompilerParams(use_tc_tiling_on_sc=False),
    scratch_types=[
        pltpu.VMEM((PER_W,), jnp.int32),
        pltpu.VMEM((NBUF * G, DIM), jnp.float32),
        pltpu.SemaphoreType.DMA,
    ],
)
def _emb_lookup(idx_hbm, table_hbm, out_hbm, idx_v, rows_v, gsem):
    wid = lax.axis_index("s") * NC + lax.axis_index("c")
    pltpu.sync_copy(idx_hbm.at[pl.ds(wid * PER_W, PER_W)], idx_v)
    base = wid * PER_W

    def block(blk, carry):
        g0 = blk * NBUF
        for b in range(NBUF):
            pltpu.async_copy(
                table_hbm.at[idx_v.at[pl.ds((g0 + b) * G, G)]],
                rows_v.at[pl.ds(b * G, G)],
                gsem,
            )
        for b in range(NBUF):
            pltpu.make_async_copy(
                table_hbm.at[idx_v.at[pl.ds((g0 + b) * G, G)]],
                rows_v.at[pl.ds(b * G, G)],
                gsem,
            ).wait()
        pltpu.sync_copy(rows_v, out_hbm.at[pl.ds(base + g0 * G, NBUF * G)])
        return carry

    lax.fori_loop(0, NBLK, block, None)


def kernel(input, weight):
    idx = input[..., 0].reshape(N)
    out = _emb_lookup(idx, weight)
    return out.reshape(SEQ, BATCH, DIM)
